# Initial kernel scaffold; baseline (speedup 1.0000x reference)
import jax
import jax.numpy as jnp
from jax.experimental import pallas as pl


def kernel(logits, m_list, target):
    raise NotImplementedError("write your pallas kernel here")


# speedup vs baseline: 5.1260x; 5.1260x over previous
import jax
import jax.numpy as jnp
from jax import lax
from jax.experimental import pallas as pl

_S = 30.0
_BLOCK = 1024


def _ldam_kernel(logits_ref, m_ref, tgt_ref, out_ref):
    i = pl.program_id(0)
    x = logits_ref[...]
    t = tgt_ref[...]
    m = m_ref[...]
    col = lax.broadcasted_iota(jnp.int32, x.shape, 1)
    onehot = col == t
    adj = x - jnp.where(onehot, _S * m, 0.0)
    mx = jnp.max(adj, axis=1, keepdims=True)
    z = jnp.sum(jnp.exp(adj - mx), axis=1, keepdims=True)
    xt = jnp.sum(jnp.where(onehot, adj, 0.0), axis=1, keepdims=True)
    part = jnp.sum(mx + jnp.log(z) - xt)

    @pl.when(i == 0)
    def _():
        out_ref[...] = jnp.zeros_like(out_ref)

    out_ref[...] += jnp.full((1, 1), part, jnp.float32)


def kernel(logits, m_list, target):
    B, C = logits.shape
    out = pl.pallas_call(
        _ldam_kernel,
        grid=(B // _BLOCK,),
        in_specs=[
            pl.BlockSpec((_BLOCK, C), lambda i: (i, 0)),
            pl.BlockSpec((1, C), lambda i: (0, 0)),
            pl.BlockSpec((_BLOCK, 1), lambda i: (i, 0)),
        ],
        out_specs=pl.BlockSpec((1, 1), lambda i: (0, 0)),
        out_shape=jax.ShapeDtypeStruct((1, 1), jnp.float32),
    )(logits, m_list.reshape(1, C), target.reshape(B, 1))
    return (out[0, 0] / B).astype(jnp.float32)
